# baseline (speedup 1.0000x reference)
import functools

import jax
import jax.numpy as jnp
from jax import lax
from jax.experimental import pallas as pl
from jax.experimental.pallas import tpu as pltpu
import jax.experimental.pallas.tpu_sc as plsc

N = 10000
E = 320000
D = 128
OUT = 128
G = 64

NC = 2
NS = 16
NW = NC * NS
EW = E // NW
K = 128
EWP = 10112
EP = NW * EWP
CH = EWP // K
NA = N + 8
RPS = N // NS

BN = 1000
NB = N // BN

_mesh = plsc.VectorSubcoreMesh(
    core_axis_name="c", subcore_axis_name="s", num_cores=NC, num_subcores=NS)


@functools.partial(
    pl.kernel,
    out_type=jax.ShapeDtypeStruct((NC, NS, RPS, D), jnp.float32),
    mesh=_mesh,
    scratch_types=[
        pltpu.VMEM((CH, K), jnp.int32),
        pltpu.VMEM((K, D), jnp.float32),
        pltpu.SemaphoreType.DMA,
        pltpu.VMEM_SHARED((NA, D), jnp.float32),
    ],
)
def _sc_degree(dst_hbm, ones_hbm, zeros_hbm, out_hbm, idx_v, ones_v, ssem,
               acc_s):
    c = lax.axis_index("c")
    s = lax.axis_index("s")
    wid = c * NS + s
    pltpu.sync_copy(zeros_hbm, acc_s.at[pl.ds(s * RPS, RPS)])
    pltpu.sync_copy(dst_hbm.at[wid], idx_v)
    pltpu.sync_copy(ones_hbm, ones_v)
    plsc.subcore_barrier()

    W = 4

    def step(i, carry):
        @pl.when(i >= W)
        def _():
            pltpu.make_async_copy(ones_v, acc_s.at[idx_v.at[i]], ssem).wait()

        pltpu.async_copy(ones_v, acc_s.at[idx_v.at[i]], ssem, add=True)
        return carry

    lax.fori_loop(0, CH, step, 0)

    def drain(i, carry):
        pltpu.make_async_copy(ones_v, acc_s.at[idx_v.at[i]], ssem).wait()
        return carry

    lax.fori_loop(0, W, drain, 0)
    plsc.subcore_barrier()
    pltpu.sync_copy(acc_s.at[pl.ds(s * RPS, RPS)], out_hbm.at[c, s])


@functools.partial(
    pl.kernel,
    out_type=jax.ShapeDtypeStruct((NC, NS, RPS, D), jnp.float32),
    mesh=_mesh,
    scratch_types=[
        pltpu.VMEM((2, K), jnp.int32),
        pltpu.VMEM((2, K), jnp.int32),
        pltpu.VMEM((2, K, D), jnp.float32),
        pltpu.SemaphoreType.DMA,
        pltpu.SemaphoreType.DMA,
        pltpu.SemaphoreType.DMA,
        pltpu.SemaphoreType.DMA,
        pltpu.VMEM_SHARED((NA, D), jnp.float32),
    ],
)
def _sc_aggregate(g_hbm, src_hbm, dst_hbm, zeros_hbm, out_hbm,
                  src_v, dst_v, rows_v, gsem, ssem, dsem, isem, acc_s):
    c = lax.axis_index("c")
    s = lax.axis_index("s")
    wid = c * NS + s
    pltpu.sync_copy(zeros_hbm, acc_s.at[pl.ds(s * RPS, RPS)])
    plsc.subcore_barrier()

    pltpu.async_copy(src_hbm.at[wid, 0], src_v.at[0], isem)
    pltpu.async_copy(src_hbm.at[wid, 1], src_v.at[1], isem)
    pltpu.async_copy(dst_hbm.at[wid, 0], dst_v.at[0], dsem)
    pltpu.make_async_copy(src_hbm.at[wid, 0], src_v.at[0], isem).wait()
    pltpu.async_copy(g_hbm.at[src_v.at[0]], rows_v.at[0], gsem)

    def step(t, carry):
        b = lax.rem(t, 2)
        nb = 1 - b

        @pl.when(t >= 1)
        def _():
            pltpu.make_async_copy(
                rows_v.at[nb], acc_s.at[dst_v.at[nb]], ssem).wait()

        @pl.when(t + 1 < CH)
        def _():
            pltpu.make_async_copy(
                src_hbm.at[wid, t + 1], src_v.at[nb], isem).wait()
            pltpu.async_copy(g_hbm.at[src_v.at[nb]], rows_v.at[nb], gsem)
            pltpu.async_copy(dst_hbm.at[wid, t + 1], dst_v.at[nb], dsem)

        pltpu.make_async_copy(g_hbm.at[src_v.at[b]], rows_v.at[b], gsem).wait()

        @pl.when(t + 2 < CH)
        def _():
            pltpu.async_copy(src_hbm.at[wid, t + 2], src_v.at[b], isem)

        pltpu.make_async_copy(dst_hbm.at[wid, t], dst_v.at[b], dsem).wait()
        pltpu.async_copy(rows_v.at[b], acc_s.at[dst_v.at[b]], ssem, add=True)
        return carry

    lax.fori_loop(0, CH, step, 0)
    pltpu.make_async_copy(rows_v.at[0], acc_s.at[dst_v.at[0]], ssem).wait()
    plsc.subcore_barrier()
    pltpu.sync_copy(acc_s.at[pl.ds(s * RPS, RPS)], out_hbm.at[c, s])


def _enc_body(x_ref, win_ref, bin_ref, w1_ref, dega_ref, degb_ref,
              g_ref, dinv_ref):
    deg = dega_ref[...][:, 0:1] + degb_ref[...][:, 0:1] + 1.0
    dinv = lax.rsqrt(deg)
    h = jnp.dot(x_ref[...], win_ref[...],
                preferred_element_type=jnp.float32) + bin_ref[...]
    hw = jnp.dot(h, w1_ref[...], preferred_element_type=jnp.float32)
    g_ref[...] = hw * dinv
    dinv_ref[...] = jnp.broadcast_to(dinv, (BN, 16))


def _post_body(a0_ref, a1_ref, g_ref, dinv_ref, b_ref, w_ref, gout_ref):
    dinv = dinv_ref[...][:, 0:1]
    h = dinv * (a0_ref[...] + a1_ref[...] + g_ref[...]) + b_ref[...]
    h = jnp.maximum(h, 0.0)
    gout_ref[...] = jnp.dot(h, w_ref[...],
                            preferred_element_type=jnp.float32) * dinv


def _final_body(a0_ref, a1_ref, g_ref, dinv_ref, b_ref, wc_ref, bc_ref,
                batch_ref, out_ref, sums, counts):
    j = pl.program_id(0)

    @pl.when(j == 0)
    def _():
        sums[...] = jnp.zeros_like(sums)
        counts[...] = jnp.zeros_like(counts)

    dinv = dinv_ref[...][:, 0:1]
    h = dinv * (a0_ref[...] + a1_ref[...] + g_ref[...]) + b_ref[...]
    y = jnp.dot(h, wc_ref[...], preferred_element_type=jnp.float32) + bc_ref[...]
    b = batch_ref[...].reshape(1, BN)
    onehot = (lax.broadcasted_iota(jnp.int32, (G, BN), 0) == b
              ).astype(jnp.float32)
    sums[...] += jnp.dot(onehot, y, preferred_element_type=jnp.float32)
    counts[...] += jnp.sum(onehot, axis=1, keepdims=True)
    out_ref[...] = sums[...] / jnp.maximum(counts[...], 1.0)


_row = lambda j: (j, 0)
_fix = lambda j: (0, 0)
_BLK = lambda shape, im: pl.BlockSpec(shape, im)


def _tc_encoder(x, w_in, b_in, w1, dega, degb):
    return pl.pallas_call(
        _enc_body,
        grid=(NB,),
        in_specs=[
            _BLK((BN, D), _row), _BLK((D, D), _fix), _BLK((1, D), _fix),
            _BLK((D, D), _fix), _BLK((BN, D), _row), _BLK((BN, D), _row),
        ],
        out_specs=[_BLK((BN, D), _row), _BLK((BN, 16), _row)],
        out_shape=[
            jax.ShapeDtypeStruct((N, D), jnp.float32),
            jax.ShapeDtypeStruct((N, 16), jnp.float32),
        ],
    )(x, w_in, b_in, w1, dega, degb)


def _tc_post(a0, a1, g, dinv, b, w):
    return pl.pallas_call(
        _post_body,
        grid=(NB,),
        in_specs=[
            _BLK((BN, D), _row), _BLK((BN, D), _row), _BLK((BN, D), _row),
            _BLK((BN, 16), _row), _BLK((1, D), _fix), _BLK((D, D), _fix),
        ],
        out_specs=_BLK((BN, D), _row),
        out_shape=jax.ShapeDtypeStruct((N, D), jnp.float32),
    )(a0, a1, g, dinv, b, w)


def _tc_final(a0, a1, g, dinv, b3, w_cls, b_cls, batch3):
    return pl.pallas_call(
        _final_body,
        grid=(NB,),
        in_specs=[
            _BLK((BN, D), _row), _BLK((BN, D), _row), _BLK((BN, D), _row),
            _BLK((BN, 16), _row), _BLK((1, D), _fix), _BLK((D, OUT), _fix),
            _BLK((1, OUT), _fix),
            pl.BlockSpec((1, 1, BN), lambda j: (j, 0, 0)),
        ],
        out_specs=_BLK((G, OUT), _fix),
        out_shape=jax.ShapeDtypeStruct((G, OUT), jnp.float32),
        scratch_shapes=[
            pltpu.VMEM((G, OUT), jnp.float32),
            pltpu.VMEM((G, 1), jnp.float32),
        ],
    )(a0, a1, g, dinv, b3, w_cls, b_cls, batch3)


def kernel(x, edge_index, batch, W_in, b_in, W1, b1, W2, b2, W3, b3,
           W_cls, b_cls):
    pad = EP - E
    src3 = jnp.concatenate(
        [edge_index[0], jnp.zeros((pad,), jnp.int32)]).reshape(NW, CH, K)
    dst3 = jnp.concatenate(
        [edge_index[1], jnp.full((pad,), N, jnp.int32)]).reshape(NW, CH, K)
    batch3 = batch.reshape(NB, 1, BN)

    onesD = jnp.ones((K, D), jnp.float32)
    zerosD = jnp.zeros((RPS, D), jnp.float32)

    deg4 = _sc_degree(dst3, onesD, zerosD)
    dega = deg4[0].reshape(N, D)
    degb = deg4[1].reshape(N, D)

    g1, dinv = _tc_encoder(x, W_in, b_in.reshape(1, D), W1, dega, degb)

    acc = _sc_aggregate(g1, src3, dst3, zerosD)
    g2 = _tc_post(acc[0].reshape(N, D), acc[1].reshape(N, D), g1, dinv,
                  b1.reshape(1, D), W2)

    acc = _sc_aggregate(g2, src3, dst3, zerosD)
    g3 = _tc_post(acc[0].reshape(N, D), acc[1].reshape(N, D), g2, dinv,
                  b2.reshape(1, D), W3)

    acc = _sc_aggregate(g3, src3, dst3, zerosD)
    pooled = _tc_final(acc[0].reshape(N, D), acc[1].reshape(N, D), g3, dinv,
                       b3.reshape(1, D), W_cls, b_cls.reshape(1, OUT), batch3)
    return pooled


# speedup vs baseline: 2.0252x; 2.0252x over previous
import functools

import jax
import jax.numpy as jnp
from jax import lax
from jax.experimental import pallas as pl
from jax.experimental.pallas import tpu as pltpu
import jax.experimental.pallas.tpu_sc as plsc

N = 10000
E = 320000
D = 128
OUT = 128
G = 64

NC = 2
NS = 16
NW = NC * NS
EW = E // NW
K = 80
CH = EW // K
RPS = N // NS

BN = 1000
NB = N // BN

_mesh = plsc.VectorSubcoreMesh(
    core_axis_name="c", subcore_axis_name="s", num_cores=NC, num_subcores=NS)


@functools.partial(
    pl.kernel,
    out_type=jax.ShapeDtypeStruct((NC, NS, RPS, D), jnp.float32),
    mesh=_mesh,
    scratch_types=[
        pltpu.VMEM((CH, K), jnp.int32),
        pltpu.VMEM((K, D), jnp.float32),
        pltpu.SemaphoreType.DMA,
        pltpu.VMEM_SHARED((N, D), jnp.float32),
    ],
)
def _sc_degree(dst_hbm, ones_hbm, zeros_hbm, out_hbm, idx_v, ones_v, ssem,
               acc_s):
    c = lax.axis_index("c")
    s = lax.axis_index("s")
    wid = c * NS + s
    pltpu.sync_copy(zeros_hbm, acc_s.at[pl.ds(s * RPS, RPS)])
    pltpu.sync_copy(dst_hbm.at[wid], idx_v)
    pltpu.sync_copy(ones_hbm, ones_v)
    plsc.subcore_barrier()

    W = 4

    def step(i, carry):
        @pl.when(i >= W)
        def _():
            pltpu.make_async_copy(ones_v, acc_s.at[idx_v.at[i]], ssem).wait()

        pltpu.async_copy(ones_v, acc_s.at[idx_v.at[i]], ssem, add=True)
        return carry

    lax.fori_loop(0, CH, step, 0)

    def drain(i, carry):
        pltpu.make_async_copy(ones_v, acc_s.at[idx_v.at[i]], ssem).wait()
        return carry

    lax.fori_loop(0, W, drain, 0)
    plsc.subcore_barrier()
    pltpu.sync_copy(acc_s.at[pl.ds(s * RPS, RPS)], out_hbm.at[c, s])


@functools.partial(
    pl.kernel,
    out_type=jax.ShapeDtypeStruct((NC, NS, RPS, D), jnp.float32),
    mesh=_mesh,
    scratch_types=[
        pltpu.VMEM((EW,), jnp.int32),
        pltpu.VMEM((2, K), jnp.int32),
        pltpu.VMEM((3, K, D), jnp.float32),
        pltpu.SemaphoreType.DMA,
        pltpu.SemaphoreType.DMA,
        pltpu.SemaphoreType.DMA,
        pltpu.VMEM_SHARED((N, D), jnp.float32),
    ],
)
def _sc_aggregate(g_hbm, src_hbm, dst_hbm, zeros_hbm, out_hbm,
                  src_v, dst_v, rows_v, gsem, ssem, dsem, acc_s):
    c = lax.axis_index("c")
    s = lax.axis_index("s")
    wid = c * NS + s
    pltpu.sync_copy(zeros_hbm, acc_s.at[pl.ds(s * RPS, RPS)])
    pltpu.sync_copy(src_hbm.at[wid], src_v)
    plsc.subcore_barrier()

    pltpu.async_copy(dst_hbm.at[wid, 0], dst_v.at[0], dsem)
    pltpu.async_copy(g_hbm.at[src_v.at[pl.ds(0, K)]], rows_v.at[0], gsem)
    pltpu.async_copy(g_hbm.at[src_v.at[pl.ds(K, K)]], rows_v.at[1], gsem)

    def step(t, carry):
        b = lax.rem(t, 3)
        db = lax.rem(t, 2)

        @pl.when(t >= 1)
        def _():
            pltpu.make_async_copy(
                rows_v.at[b], acc_s.at[dst_v.at[db]], ssem).wait()

        @pl.when(t + 1 < CH)
        def _():
            pltpu.async_copy(dst_hbm.at[wid, t + 1], dst_v.at[1 - db], dsem)

        @pl.when(t + 2 < CH)
        def _():
            pltpu.async_copy(
                g_hbm.at[src_v.at[pl.ds((t + 2) * K, K)]],
                rows_v.at[lax.rem(t + 2, 3)], gsem)

        pltpu.make_async_copy(
            g_hbm.at[src_v.at[pl.ds(t * K, K)]], rows_v.at[b], gsem).wait()
        pltpu.make_async_copy(dst_hbm.at[wid, t], dst_v.at[db], dsem).wait()
        pltpu.async_copy(rows_v.at[b], acc_s.at[dst_v.at[db]], ssem, add=True)
        return carry

    lax.fori_loop(0, CH, step, 0)
    pltpu.make_async_copy(rows_v.at[0], acc_s.at[dst_v.at[0]], ssem).wait()
    plsc.subcore_barrier()
    pltpu.sync_copy(acc_s.at[pl.ds(s * RPS, RPS)], out_hbm.at[c, s])


def _enc_body(x_ref, win_ref, bin_ref, w1_ref, dega_ref, degb_ref,
              g_ref, dinv_ref):
    deg = dega_ref[...][:, 0:1] + degb_ref[...][:, 0:1] + 1.0
    dinv = lax.rsqrt(deg)
    h = jnp.dot(x_ref[...], win_ref[...],
                preferred_element_type=jnp.float32) + bin_ref[...]
    hw = jnp.dot(h, w1_ref[...], preferred_element_type=jnp.float32)
    g_ref[...] = hw * dinv
    dinv_ref[...] = jnp.broadcast_to(dinv, (BN, 16))


def _post_body(a0_ref, a1_ref, g_ref, dinv_ref, b_ref, w_ref, gout_ref):
    dinv = dinv_ref[...][:, 0:1]
    h = dinv * (a0_ref[...] + a1_ref[...] + g_ref[...]) + b_ref[...]
    h = jnp.maximum(h, 0.0)
    gout_ref[...] = jnp.dot(h, w_ref[...],
                            preferred_element_type=jnp.float32) * dinv


def _final_body(a0_ref, a1_ref, g_ref, dinv_ref, b_ref, wc_ref, bc_ref,
                batch_ref, out_ref, sums, counts):
    j = pl.program_id(0)

    @pl.when(j == 0)
    def _():
        sums[...] = jnp.zeros_like(sums)
        counts[...] = jnp.zeros_like(counts)

    dinv = dinv_ref[...][:, 0:1]
    h = dinv * (a0_ref[...] + a1_ref[...] + g_ref[...]) + b_ref[...]
    y = jnp.dot(h, wc_ref[...], preferred_element_type=jnp.float32) + bc_ref[...]
    b = batch_ref[...].reshape(1, BN)
    onehot = (lax.broadcasted_iota(jnp.int32, (G, BN), 0) == b
              ).astype(jnp.float32)
    sums[...] += jnp.dot(onehot, y, preferred_element_type=jnp.float32)
    counts[...] += jnp.sum(onehot, axis=1, keepdims=True)
    out_ref[...] = sums[...] / jnp.maximum(counts[...], 1.0)


_row = lambda j: (j, 0)
_fix = lambda j: (0, 0)
_BLK = lambda shape, im: pl.BlockSpec(shape, im)


def _tc_encoder(x, w_in, b_in, w1, dega, degb):
    return pl.pallas_call(
        _enc_body,
        grid=(NB,),
        in_specs=[
            _BLK((BN, D), _row), _BLK((D, D), _fix), _BLK((1, D), _fix),
            _BLK((D, D), _fix), _BLK((BN, D), _row), _BLK((BN, D), _row),
        ],
        out_specs=[_BLK((BN, D), _row), _BLK((BN, 16), _row)],
        out_shape=[
            jax.ShapeDtypeStruct((N, D), jnp.float32),
            jax.ShapeDtypeStruct((N, 16), jnp.float32),
        ],
    )(x, w_in, b_in, w1, dega, degb)


def _tc_post(a0, a1, g, dinv, b, w):
    return pl.pallas_call(
        _post_body,
        grid=(NB,),
        in_specs=[
            _BLK((BN, D), _row), _BLK((BN, D), _row), _BLK((BN, D), _row),
            _BLK((BN, 16), _row), _BLK((1, D), _fix), _BLK((D, D), _fix),
        ],
        out_specs=_BLK((BN, D), _row),
        out_shape=jax.ShapeDtypeStruct((N, D), jnp.float32),
    )(a0, a1, g, dinv, b, w)


def _tc_final(a0, a1, g, dinv, b3, w_cls, b_cls, batch3):
    return pl.pallas_call(
        _final_body,
        grid=(NB,),
        in_specs=[
            _BLK((BN, D), _row), _BLK((BN, D), _row), _BLK((BN, D), _row),
            _BLK((BN, 16), _row), _BLK((1, D), _fix), _BLK((D, OUT), _fix),
            _BLK((1, OUT), _fix),
            pl.BlockSpec((1, 1, BN), lambda j: (j, 0, 0)),
        ],
        out_specs=_BLK((G, OUT), _fix),
        out_shape=jax.ShapeDtypeStruct((G, OUT), jnp.float32),
        scratch_shapes=[
            pltpu.VMEM((G, OUT), jnp.float32),
            pltpu.VMEM((G, 1), jnp.float32),
        ],
    )(a0, a1, g, dinv, b3, w_cls, b_cls, batch3)


def kernel(x, edge_index, batch, W_in, b_in, W1, b1, W2, b2, W3, b3,
           W_cls, b_cls):
    src2 = edge_index[0].reshape(NW, EW)
    dst3 = edge_index[1].reshape(NW, CH, K)
    batch3 = batch.reshape(NB, 1, BN)

    onesD = jnp.ones((K, D), jnp.float32)
    zerosD = jnp.zeros((RPS, D), jnp.float32)

    deg4 = _sc_degree(dst3, onesD, zerosD)
    dega = deg4[0].reshape(N, D)
    degb = deg4[1].reshape(N, D)

    g1, dinv = _tc_encoder(x, W_in, b_in.reshape(1, D), W1, dega, degb)

    acc = _sc_aggregate(g1, src2, dst3, zerosD)
    g2 = _tc_post(acc[0].reshape(N, D), acc[1].reshape(N, D), g1, dinv,
                  b1.reshape(1, D), W2)

    acc = _sc_aggregate(g2, src2, dst3, zerosD)
    g3 = _tc_post(acc[0].reshape(N, D), acc[1].reshape(N, D), g2, dinv,
                  b2.reshape(1, D), W3)

    acc = _sc_aggregate(g3, src2, dst3, zerosD)
    pooled = _tc_final(acc[0].reshape(N, D), acc[1].reshape(N, D), g3, dinv,
                       b3.reshape(1, D), W_cls, b_cls.reshape(1, OUT), batch3)
    return pooled


# speedup vs baseline: 2.0654x; 1.0199x over previous
import functools

import jax
import jax.numpy as jnp
from jax import lax
from jax.experimental import pallas as pl
from jax.experimental.pallas import tpu as pltpu
import jax.experimental.pallas.tpu_sc as plsc

N = 10000
E = 320000
D = 128
OUT = 128
G = 64

NC = 2
NS = 16
NW = NC * NS
EW = E // NW
K = 80
CH = EW // K
RPS = N // NS

BN = 2000
NB = N // BN

_mesh = plsc.VectorSubcoreMesh(
    core_axis_name="c", subcore_axis_name="s", num_cores=NC, num_subcores=NS)


@functools.partial(
    pl.kernel,
    out_type=jax.ShapeDtypeStruct((NC, NS, RPS, D), jnp.float32),
    mesh=_mesh,
    scratch_types=[
        pltpu.VMEM((CH, K), jnp.int32),
        pltpu.VMEM((K, D), jnp.float32),
        pltpu.SemaphoreType.DMA,
        pltpu.VMEM_SHARED((N, D), jnp.float32),
    ],
)
def _sc_degree(dst_hbm, ones_hbm, zeros_hbm, out_hbm, idx_v, ones_v, ssem,
               acc_s):
    c = lax.axis_index("c")
    s = lax.axis_index("s")
    wid = c * NS + s
    pltpu.sync_copy(zeros_hbm, acc_s.at[pl.ds(s * RPS, RPS)])
    pltpu.sync_copy(dst_hbm.at[wid], idx_v)
    pltpu.sync_copy(ones_hbm, ones_v)
    plsc.subcore_barrier()

    W = 4

    def step(i, carry):
        @pl.when(i >= W)
        def _():
            pltpu.make_async_copy(ones_v, acc_s.at[idx_v.at[i]], ssem).wait()

        pltpu.async_copy(ones_v, acc_s.at[idx_v.at[i]], ssem, add=True)
        return carry

    lax.fori_loop(0, CH, step, 0)

    def drain(i, carry):
        pltpu.make_async_copy(ones_v, acc_s.at[idx_v.at[i]], ssem).wait()
        return carry

    lax.fori_loop(0, W, drain, 0)
    plsc.subcore_barrier()
    pltpu.sync_copy(acc_s.at[pl.ds(s * RPS, RPS)], out_hbm.at[c, s])


@functools.partial(
    pl.kernel,
    out_type=jax.ShapeDtypeStruct((NC, NS, RPS, D), jnp.float32),
    mesh=_mesh,
    scratch_types=[
        pltpu.VMEM((EW,), jnp.int32),
        pltpu.VMEM((2, K), jnp.int32),
        pltpu.VMEM((3, K, D), jnp.float32),
        pltpu.SemaphoreType.DMA,
        pltpu.SemaphoreType.DMA,
        pltpu.SemaphoreType.DMA,
        pltpu.VMEM_SHARED((N, D), jnp.float32),
    ],
)
def _sc_aggregate(g_hbm, src_hbm, dst_hbm, zeros_hbm, out_hbm,
                  src_v, dst_v, rows_v, gsem, ssem, dsem, acc_s):
    c = lax.axis_index("c")
    s = lax.axis_index("s")
    wid = c * NS + s
    pltpu.sync_copy(zeros_hbm, acc_s.at[pl.ds(s * RPS, RPS)])
    pltpu.sync_copy(src_hbm.at[wid], src_v)
    plsc.subcore_barrier()

    pltpu.async_copy(dst_hbm.at[wid, 0], dst_v.at[0], dsem)
    pltpu.async_copy(g_hbm.at[src_v.at[pl.ds(0, K)]], rows_v.at[0], gsem)
    pltpu.async_copy(g_hbm.at[src_v.at[pl.ds(K, K)]], rows_v.at[1], gsem)

    def step(t, carry):
        b = lax.rem(t, 3)
        db = lax.rem(t, 2)

        @pl.when(t >= 1)
        def _():
            pltpu.make_async_copy(
                rows_v.at[b], acc_s.at[dst_v.at[db]], ssem).wait()

        @pl.when(t + 1 < CH)
        def _():
            pltpu.async_copy(dst_hbm.at[wid, t + 1], dst_v.at[1 - db], dsem)

        @pl.when(t + 2 < CH)
        def _():
            pltpu.async_copy(
                g_hbm.at[src_v.at[pl.ds((t + 2) * K, K)]],
                rows_v.at[lax.rem(t + 2, 3)], gsem)

        pltpu.make_async_copy(
            g_hbm.at[src_v.at[pl.ds(t * K, K)]], rows_v.at[b], gsem).wait()
        pltpu.make_async_copy(dst_hbm.at[wid, t], dst_v.at[db], dsem).wait()
        pltpu.async_copy(rows_v.at[b], acc_s.at[dst_v.at[db]], ssem, add=True)
        return carry

    lax.fori_loop(0, CH, step, 0)
    pltpu.make_async_copy(rows_v.at[0], acc_s.at[dst_v.at[0]], ssem).wait()
    plsc.subcore_barrier()
    pltpu.sync_copy(acc_s.at[pl.ds(s * RPS, RPS)], out_hbm.at[c, s])


def _enc_body(x_ref, win_ref, bin_ref, w1_ref, dega_ref, degb_ref,
              g_ref, dinv_ref):
    deg = dega_ref[...][:, 0:1] + degb_ref[...][:, 0:1] + 1.0
    dinv = lax.rsqrt(deg)
    h = jnp.dot(x_ref[...], win_ref[...],
                preferred_element_type=jnp.float32) + bin_ref[...]
    hw = jnp.dot(h, w1_ref[...], preferred_element_type=jnp.float32)
    g_ref[...] = hw * dinv
    dinv_ref[...] = jnp.broadcast_to(dinv, (BN, 16))


def _post_body(a0_ref, a1_ref, g_ref, dinv_ref, b_ref, w_ref, gout_ref):
    dinv = dinv_ref[...][:, 0:1]
    h = dinv * (a0_ref[...] + a1_ref[...] + g_ref[...]) + b_ref[...]
    h = jnp.maximum(h, 0.0)
    gout_ref[...] = jnp.dot(h, w_ref[...],
                            preferred_element_type=jnp.float32) * dinv


def _final_body(a0_ref, a1_ref, g_ref, dinv_ref, b_ref, wc_ref, bc_ref,
                batch_ref, out_ref, sums, counts):
    j = pl.program_id(0)

    @pl.when(j == 0)
    def _():
        sums[...] = jnp.zeros_like(sums)
        counts[...] = jnp.zeros_like(counts)

    dinv = dinv_ref[...][:, 0:1]
    h = dinv * (a0_ref[...] + a1_ref[...] + g_ref[...]) + b_ref[...]
    y = jnp.dot(h, wc_ref[...], preferred_element_type=jnp.float32) + bc_ref[...]
    b = batch_ref[...].reshape(1, BN)
    onehot = (lax.broadcasted_iota(jnp.int32, (G, BN), 0) == b
              ).astype(jnp.float32)
    sums[...] += jnp.dot(onehot, y, preferred_element_type=jnp.float32)
    counts[...] += jnp.sum(onehot, axis=1, keepdims=True)
    out_ref[...] = sums[...] / jnp.maximum(counts[...], 1.0)


_row = lambda j: (j, 0)
_fix = lambda j: (0, 0)
_BLK = lambda shape, im: pl.BlockSpec(shape, im)


def _tc_encoder(x, w_in, b_in, w1, dega, degb):
    return pl.pallas_call(
        _enc_body,
        grid=(NB,),
        in_specs=[
            _BLK((BN, D), _row), _BLK((D, D), _fix), _BLK((1, D), _fix),
            _BLK((D, D), _fix), _BLK((BN, D), _row), _BLK((BN, D), _row),
        ],
        out_specs=[_BLK((BN, D), _row), _BLK((BN, 16), _row)],
        out_shape=[
            jax.ShapeDtypeStruct((N, D), jnp.float32),
            jax.ShapeDtypeStruct((N, 16), jnp.float32),
        ],
    )(x, w_in, b_in, w1, dega, degb)


def _tc_post(a0, a1, g, dinv, b, w):
    return pl.pallas_call(
        _post_body,
        grid=(NB,),
        in_specs=[
            _BLK((BN, D), _row), _BLK((BN, D), _row), _BLK((BN, D), _row),
            _BLK((BN, 16), _row), _BLK((1, D), _fix), _BLK((D, D), _fix),
        ],
        out_specs=_BLK((BN, D), _row),
        out_shape=jax.ShapeDtypeStruct((N, D), jnp.float32),
    )(a0, a1, g, dinv, b, w)


def _tc_final(a0, a1, g, dinv, b3, w_cls, b_cls, batch3):
    return pl.pallas_call(
        _final_body,
        grid=(NB,),
        in_specs=[
            _BLK((BN, D), _row), _BLK((BN, D), _row), _BLK((BN, D), _row),
            _BLK((BN, 16), _row), _BLK((1, D), _fix), _BLK((D, OUT), _fix),
            _BLK((1, OUT), _fix),
            pl.BlockSpec((1, 1, BN), lambda j: (j, 0, 0)),
        ],
        out_specs=_BLK((G, OUT), _fix),
        out_shape=jax.ShapeDtypeStruct((G, OUT), jnp.float32),
        scratch_shapes=[
            pltpu.VMEM((G, OUT), jnp.float32),
            pltpu.VMEM((G, 1), jnp.float32),
        ],
    )(a0, a1, g, dinv, b3, w_cls, b_cls, batch3)


def kernel(x, edge_index, batch, W_in, b_in, W1, b1, W2, b2, W3, b3,
           W_cls, b_cls):
    src2 = edge_index[0].reshape(NW, EW)
    dst3 = edge_index[1].reshape(NW, CH, K)
    batch3 = batch.reshape(NB, 1, BN)

    onesD = jnp.ones((K, D), jnp.float32)
    zerosD = jnp.zeros((RPS, D), jnp.float32)

    deg4 = _sc_degree(dst3, onesD, zerosD)
    dega = deg4[0].reshape(N, D)
    degb = deg4[1].reshape(N, D)

    g1, dinv = _tc_encoder(x, W_in, b_in.reshape(1, D), W1, dega, degb)

    acc = _sc_aggregate(g1, src2, dst3, zerosD)
    g2 = _tc_post(acc[0].reshape(N, D), acc[1].reshape(N, D), g1, dinv,
                  b1.reshape(1, D), W2)

    acc = _sc_aggregate(g2, src2, dst3, zerosD)
    g3 = _tc_post(acc[0].reshape(N, D), acc[1].reshape(N, D), g2, dinv,
                  b2.reshape(1, D), W3)

    acc = _sc_aggregate(g3, src2, dst3, zerosD)
    pooled = _tc_final(acc[0].reshape(N, D), acc[1].reshape(N, D), g3, dinv,
                       b3.reshape(1, D), W_cls, b_cls.reshape(1, OUT), batch3)
    return pooled


# speedup vs baseline: 2.0894x; 1.0116x over previous
import functools

import jax
import jax.numpy as jnp
from jax import lax
from jax.experimental import pallas as pl
from jax.experimental.pallas import tpu as pltpu
import jax.experimental.pallas.tpu_sc as plsc

N = 10000
E = 320000
D = 128
OUT = 128
G = 64

NC = 2
NS = 16
NW = NC * NS
EW = E // NW
K = 80
CH = EW // K
RPS = N // NS

BN = 5000
NB = N // BN

_mesh = plsc.VectorSubcoreMesh(
    core_axis_name="c", subcore_axis_name="s", num_cores=NC, num_subcores=NS)


@functools.partial(
    pl.kernel,
    out_type=jax.ShapeDtypeStruct((NC, NS, RPS, D), jnp.float32),
    mesh=_mesh,
    scratch_types=[
        pltpu.VMEM((CH, K), jnp.int32),
        pltpu.VMEM((K, D), jnp.float32),
        pltpu.SemaphoreType.DMA,
        pltpu.VMEM_SHARED((N, D), jnp.float32),
    ],
)
def _sc_degree(dst_hbm, ones_hbm, zeros_hbm, out_hbm, idx_v, ones_v, ssem,
               acc_s):
    c = lax.axis_index("c")
    s = lax.axis_index("s")
    wid = c * NS + s
    pltpu.sync_copy(zeros_hbm, acc_s.at[pl.ds(s * RPS, RPS)])
    pltpu.sync_copy(dst_hbm.at[wid], idx_v)
    pltpu.sync_copy(ones_hbm, ones_v)
    plsc.subcore_barrier()

    W = 4

    def step(i, carry):
        @pl.when(i >= W)
        def _():
            pltpu.make_async_copy(ones_v, acc_s.at[idx_v.at[i]], ssem).wait()

        pltpu.async_copy(ones_v, acc_s.at[idx_v.at[i]], ssem, add=True)
        return carry

    lax.fori_loop(0, CH, step, 0)

    def drain(i, carry):
        pltpu.make_async_copy(ones_v, acc_s.at[idx_v.at[i]], ssem).wait()
        return carry

    lax.fori_loop(0, W, drain, 0)
    plsc.subcore_barrier()
    pltpu.sync_copy(acc_s.at[pl.ds(s * RPS, RPS)], out_hbm.at[c, s])


@functools.partial(
    pl.kernel,
    out_type=jax.ShapeDtypeStruct((NC, NS, RPS, D), jnp.float32),
    mesh=_mesh,
    scratch_types=[
        pltpu.VMEM((EW,), jnp.int32),
        pltpu.VMEM((2, K), jnp.int32),
        pltpu.VMEM((3, K, D), jnp.float32),
        pltpu.SemaphoreType.DMA,
        pltpu.SemaphoreType.DMA,
        pltpu.SemaphoreType.DMA,
        pltpu.VMEM_SHARED((N, D), jnp.float32),
    ],
)
def _sc_aggregate(g_hbm, src_hbm, dst_hbm, zeros_hbm, out_hbm,
                  src_v, dst_v, rows_v, gsem, ssem, dsem, acc_s):
    c = lax.axis_index("c")
    s = lax.axis_index("s")
    wid = c * NS + s
    pltpu.sync_copy(zeros_hbm, acc_s.at[pl.ds(s * RPS, RPS)])
    pltpu.sync_copy(src_hbm.at[wid], src_v)
    plsc.subcore_barrier()

    pltpu.async_copy(dst_hbm.at[wid, 0], dst_v.at[0], dsem)
    pltpu.async_copy(g_hbm.at[src_v.at[pl.ds(0, K)]], rows_v.at[0], gsem)
    pltpu.async_copy(g_hbm.at[src_v.at[pl.ds(K, K)]], rows_v.at[1], gsem)

    def step(t, carry):
        b = lax.rem(t, 3)
        db = lax.rem(t, 2)

        @pl.when(t >= 1)
        def _():
            pltpu.make_async_copy(
                rows_v.at[b], acc_s.at[dst_v.at[db]], ssem).wait()

        @pl.when(t + 1 < CH)
        def _():
            pltpu.async_copy(dst_hbm.at[wid, t + 1], dst_v.at[1 - db], dsem)

        @pl.when(t + 2 < CH)
        def _():
            pltpu.async_copy(
                g_hbm.at[src_v.at[pl.ds((t + 2) * K, K)]],
                rows_v.at[lax.rem(t + 2, 3)], gsem)

        pltpu.make_async_copy(
            g_hbm.at[src_v.at[pl.ds(t * K, K)]], rows_v.at[b], gsem).wait()
        pltpu.make_async_copy(dst_hbm.at[wid, t], dst_v.at[db], dsem).wait()
        pltpu.async_copy(rows_v.at[b], acc_s.at[dst_v.at[db]], ssem, add=True)
        return carry

    lax.fori_loop(0, CH, step, 0)
    pltpu.make_async_copy(rows_v.at[0], acc_s.at[dst_v.at[0]], ssem).wait()
    plsc.subcore_barrier()
    pltpu.sync_copy(acc_s.at[pl.ds(s * RPS, RPS)], out_hbm.at[c, s])


def _enc_body(x_ref, win_ref, bin_ref, w1_ref, dega_ref, degb_ref,
              g_ref, dinv_ref):
    deg = dega_ref[...][:, 0:1] + degb_ref[...][:, 0:1] + 1.0
    dinv = lax.rsqrt(deg)
    h = jnp.dot(x_ref[...], win_ref[...],
                preferred_element_type=jnp.float32) + bin_ref[...]
    hw = jnp.dot(h, w1_ref[...], preferred_element_type=jnp.float32)
    g_ref[...] = hw * dinv
    dinv_ref[...] = jnp.broadcast_to(dinv, (BN, 16))


def _post_body(a0_ref, a1_ref, g_ref, dinv_ref, b_ref, w_ref, gout_ref):
    dinv = dinv_ref[...][:, 0:1]
    h = dinv * (a0_ref[...] + a1_ref[...] + g_ref[...]) + b_ref[...]
    h = jnp.maximum(h, 0.0)
    gout_ref[...] = jnp.dot(h, w_ref[...],
                            preferred_element_type=jnp.float32) * dinv


def _final_body(a0_ref, a1_ref, g_ref, dinv_ref, b_ref, wc_ref, bc_ref,
                batch_ref, out_ref, sums, counts):
    j = pl.program_id(0)

    @pl.when(j == 0)
    def _():
        sums[...] = jnp.zeros_like(sums)
        counts[...] = jnp.zeros_like(counts)

    dinv = dinv_ref[...][:, 0:1]
    h = dinv * (a0_ref[...] + a1_ref[...] + g_ref[...]) + b_ref[...]
    y = jnp.dot(h, wc_ref[...], preferred_element_type=jnp.float32) + bc_ref[...]
    b = batch_ref[...].reshape(1, BN)
    onehot = (lax.broadcasted_iota(jnp.int32, (G, BN), 0) == b
              ).astype(jnp.float32)
    sums[...] += jnp.dot(onehot, y, preferred_element_type=jnp.float32)
    counts[...] += jnp.sum(onehot, axis=1, keepdims=True)
    out_ref[...] = sums[...] / jnp.maximum(counts[...], 1.0)


_row = lambda j: (j, 0)
_fix = lambda j: (0, 0)
_BLK = lambda shape, im: pl.BlockSpec(shape, im)


def _tc_encoder(x, w_in, b_in, w1, dega, degb):
    return pl.pallas_call(
        _enc_body,
        grid=(NB,),
        in_specs=[
            _BLK((BN, D), _row), _BLK((D, D), _fix), _BLK((1, D), _fix),
            _BLK((D, D), _fix), _BLK((BN, D), _row), _BLK((BN, D), _row),
        ],
        out_specs=[_BLK((BN, D), _row), _BLK((BN, 16), _row)],
        out_shape=[
            jax.ShapeDtypeStruct((N, D), jnp.float32),
            jax.ShapeDtypeStruct((N, 16), jnp.float32),
        ],
    )(x, w_in, b_in, w1, dega, degb)


def _tc_post(a0, a1, g, dinv, b, w):
    return pl.pallas_call(
        _post_body,
        grid=(NB,),
        in_specs=[
            _BLK((BN, D), _row), _BLK((BN, D), _row), _BLK((BN, D), _row),
            _BLK((BN, 16), _row), _BLK((1, D), _fix), _BLK((D, D), _fix),
        ],
        out_specs=_BLK((BN, D), _row),
        out_shape=jax.ShapeDtypeStruct((N, D), jnp.float32),
    )(a0, a1, g, dinv, b, w)


def _tc_final(a0, a1, g, dinv, b3, w_cls, b_cls, batch3):
    return pl.pallas_call(
        _final_body,
        grid=(NB,),
        in_specs=[
            _BLK((BN, D), _row), _BLK((BN, D), _row), _BLK((BN, D), _row),
            _BLK((BN, 16), _row), _BLK((1, D), _fix), _BLK((D, OUT), _fix),
            _BLK((1, OUT), _fix),
            pl.BlockSpec((1, 1, BN), lambda j: (j, 0, 0)),
        ],
        out_specs=_BLK((G, OUT), _fix),
        out_shape=jax.ShapeDtypeStruct((G, OUT), jnp.float32),
        scratch_shapes=[
            pltpu.VMEM((G, OUT), jnp.float32),
            pltpu.VMEM((G, 1), jnp.float32),
        ],
    )(a0, a1, g, dinv, b3, w_cls, b_cls, batch3)


def kernel(x, edge_index, batch, W_in, b_in, W1, b1, W2, b2, W3, b3,
           W_cls, b_cls):
    src2 = edge_index[0].reshape(NW, EW)
    dst3 = edge_index[1].reshape(NW, CH, K)
    batch3 = batch.reshape(NB, 1, BN)

    onesD = jnp.ones((K, D), jnp.float32)
    zerosD = jnp.zeros((RPS, D), jnp.float32)

    deg4 = _sc_degree(dst3, onesD, zerosD)
    dega = deg4[0].reshape(N, D)
    degb = deg4[1].reshape(N, D)

    g1, dinv = _tc_encoder(x, W_in, b_in.reshape(1, D), W1, dega, degb)

    acc = _sc_aggregate(g1, src2, dst3, zerosD)
    g2 = _tc_post(acc[0].reshape(N, D), acc[1].reshape(N, D), g1, dinv,
                  b1.reshape(1, D), W2)

    acc = _sc_aggregate(g2, src2, dst3, zerosD)
    g3 = _tc_post(acc[0].reshape(N, D), acc[1].reshape(N, D), g2, dinv,
                  b2.reshape(1, D), W3)

    acc = _sc_aggregate(g3, src2, dst3, zerosD)
    pooled = _tc_final(acc[0].reshape(N, D), acc[1].reshape(N, D), g3, dinv,
                       b3.reshape(1, D), W_cls, b_cls.reshape(1, OUT), batch3)
    return pooled


# speedup vs baseline: 2.1108x; 1.0103x over previous
import functools

import jax
import jax.numpy as jnp
from jax import lax
from jax.experimental import pallas as pl
from jax.experimental.pallas import tpu as pltpu
import jax.experimental.pallas.tpu_sc as plsc

N = 10000
E = 320000
D = 128
OUT = 128
G = 64

NC = 2
NS = 16
NW = NC * NS
EW = E // NW
K = 80
CH = EW // K
RPS = N // NS

BN = 5000
NB = N // BN

_mesh = plsc.VectorSubcoreMesh(
    core_axis_name="c", subcore_axis_name="s", num_cores=NC, num_subcores=NS)


@functools.partial(
    pl.kernel,
    out_type=jax.ShapeDtypeStruct((NC, NS, RPS, D), jnp.float32),
    mesh=_mesh,
    scratch_types=[
        pltpu.VMEM((CH, K), jnp.int32),
        pltpu.VMEM((K, D), jnp.float32),
        pltpu.SemaphoreType.DMA,
        pltpu.VMEM_SHARED((N, D), jnp.float32),
    ],
)
def _sc_degree(dst_hbm, ones_hbm, zeros_hbm, out_hbm, idx_v, ones_v, ssem,
               acc_s):
    c = lax.axis_index("c")
    s = lax.axis_index("s")
    wid = c * NS + s
    zdesc = pltpu.async_copy(zeros_hbm, acc_s.at[pl.ds(s * RPS, RPS)], ssem)
    pltpu.sync_copy(dst_hbm.at[wid], idx_v)
    pltpu.sync_copy(ones_hbm, ones_v)
    zdesc.wait()
    plsc.subcore_barrier()

    W = 4

    def step(i, carry):
        @pl.when(i >= W)
        def _():
            pltpu.make_async_copy(ones_v, acc_s.at[idx_v.at[i]], ssem).wait()

        pltpu.async_copy(ones_v, acc_s.at[idx_v.at[i]], ssem, add=True)
        return carry

    lax.fori_loop(0, CH, step, 0)

    def drain(i, carry):
        pltpu.make_async_copy(ones_v, acc_s.at[idx_v.at[i]], ssem).wait()
        return carry

    lax.fori_loop(0, W, drain, 0)
    plsc.subcore_barrier()
    pltpu.sync_copy(acc_s.at[pl.ds(s * RPS, RPS)], out_hbm.at[c, s])


@functools.partial(
    pl.kernel,
    out_type=jax.ShapeDtypeStruct((NC, NS, RPS, D), jnp.float32),
    mesh=_mesh,
    scratch_types=[
        pltpu.VMEM((EW,), jnp.int32),
        pltpu.VMEM((2, K), jnp.int32),
        pltpu.VMEM((3, K, D), jnp.float32),
        pltpu.SemaphoreType.DMA,
        pltpu.SemaphoreType.DMA,
        pltpu.SemaphoreType.DMA,
        pltpu.VMEM_SHARED((N, D), jnp.float32),
    ],
)
def _sc_aggregate(g_hbm, src_hbm, dst_hbm, zeros_hbm, out_hbm,
                  src_v, dst_v, rows_v, gsem, ssem, dsem, acc_s):
    c = lax.axis_index("c")
    s = lax.axis_index("s")
    wid = c * NS + s
    zdesc = pltpu.async_copy(zeros_hbm, acc_s.at[pl.ds(s * RPS, RPS)], ssem)
    pltpu.sync_copy(src_hbm.at[wid], src_v)

    pltpu.async_copy(dst_hbm.at[wid, 0], dst_v.at[0], dsem)
    pltpu.async_copy(g_hbm.at[src_v.at[pl.ds(0, K)]], rows_v.at[0], gsem)
    pltpu.async_copy(g_hbm.at[src_v.at[pl.ds(K, K)]], rows_v.at[1], gsem)
    zdesc.wait()
    plsc.subcore_barrier()

    def step(t, carry):
        b = lax.rem(t, 3)
        db = lax.rem(t, 2)

        @pl.when(t >= 1)
        def _():
            pltpu.make_async_copy(
                rows_v.at[b], acc_s.at[dst_v.at[db]], ssem).wait()

        @pl.when(t + 1 < CH)
        def _():
            pltpu.async_copy(dst_hbm.at[wid, t + 1], dst_v.at[1 - db], dsem)

        @pl.when(t + 2 < CH)
        def _():
            pltpu.async_copy(
                g_hbm.at[src_v.at[pl.ds((t + 2) * K, K)]],
                rows_v.at[lax.rem(t + 2, 3)], gsem)

        pltpu.make_async_copy(
            g_hbm.at[src_v.at[pl.ds(t * K, K)]], rows_v.at[b], gsem).wait()
        pltpu.make_async_copy(dst_hbm.at[wid, t], dst_v.at[db], dsem).wait()
        pltpu.async_copy(rows_v.at[b], acc_s.at[dst_v.at[db]], ssem, add=True)
        return carry

    lax.fori_loop(0, CH, step, 0)
    pltpu.make_async_copy(rows_v.at[0], acc_s.at[dst_v.at[0]], ssem).wait()
    plsc.subcore_barrier()
    pltpu.sync_copy(acc_s.at[pl.ds(s * RPS, RPS)], out_hbm.at[c, s])


def _enc_body(x_ref, win_ref, bin_ref, w1_ref, dega_ref, degb_ref,
              g_ref, dinv_ref):
    deg = dega_ref[...][:, 0:1] + degb_ref[...][:, 0:1] + 1.0
    dinv = lax.rsqrt(deg)
    h = jnp.dot(x_ref[...], win_ref[...],
                preferred_element_type=jnp.float32) + bin_ref[...]
    hw = jnp.dot(h, w1_ref[...], preferred_element_type=jnp.float32)
    g_ref[...] = hw * dinv
    dinv_ref[...] = jnp.broadcast_to(dinv, (BN, 16))


def _post_body(a0_ref, a1_ref, g_ref, dinv_ref, b_ref, w_ref, gout_ref):
    dinv = dinv_ref[...][:, 0:1]
    h = dinv * (a0_ref[...] + a1_ref[...] + g_ref[...]) + b_ref[...]
    h = jnp.maximum(h, 0.0)
    gout_ref[...] = jnp.dot(h, w_ref[...],
                            preferred_element_type=jnp.float32) * dinv


def _final_body(a0_ref, a1_ref, g_ref, dinv_ref, b_ref, wc_ref, bc_ref,
                batch_ref, out_ref, sums, counts):
    j = pl.program_id(0)

    @pl.when(j == 0)
    def _():
        sums[...] = jnp.zeros_like(sums)
        counts[...] = jnp.zeros_like(counts)

    dinv = dinv_ref[...][:, 0:1]
    h = dinv * (a0_ref[...] + a1_ref[...] + g_ref[...]) + b_ref[...]
    y = jnp.dot(h, wc_ref[...], preferred_element_type=jnp.float32) + bc_ref[...]
    b = batch_ref[...].reshape(1, BN)
    onehot = (lax.broadcasted_iota(jnp.int32, (G, BN), 0) == b
              ).astype(jnp.float32)
    sums[...] += jnp.dot(onehot, y, preferred_element_type=jnp.float32)
    counts[...] += jnp.sum(onehot, axis=1, keepdims=True)
    out_ref[...] = sums[...] / jnp.maximum(counts[...], 1.0)


_row = lambda j: (j, 0)
_fix = lambda j: (0, 0)
_BLK = lambda shape, im: pl.BlockSpec(shape, im)


def _tc_encoder(x, w_in, b_in, w1, dega, degb):
    return pl.pallas_call(
        _enc_body,
        grid=(NB,),
        in_specs=[
            _BLK((BN, D), _row), _BLK((D, D), _fix), _BLK((1, D), _fix),
            _BLK((D, D), _fix), _BLK((BN, D), _row), _BLK((BN, D), _row),
        ],
        out_specs=[_BLK((BN, D), _row), _BLK((BN, 16), _row)],
        out_shape=[
            jax.ShapeDtypeStruct((N, D), jnp.float32),
            jax.ShapeDtypeStruct((N, 16), jnp.float32),
        ],
    )(x, w_in, b_in, w1, dega, degb)


def _tc_post(a0, a1, g, dinv, b, w):
    return pl.pallas_call(
        _post_body,
        grid=(NB,),
        in_specs=[
            _BLK((BN, D), _row), _BLK((BN, D), _row), _BLK((BN, D), _row),
            _BLK((BN, 16), _row), _BLK((1, D), _fix), _BLK((D, D), _fix),
        ],
        out_specs=_BLK((BN, D), _row),
        out_shape=jax.ShapeDtypeStruct((N, D), jnp.float32),
    )(a0, a1, g, dinv, b, w)


def _tc_final(a0, a1, g, dinv, b3, w_cls, b_cls, batch3):
    return pl.pallas_call(
        _final_body,
        grid=(NB,),
        in_specs=[
            _BLK((BN, D), _row), _BLK((BN, D), _row), _BLK((BN, D), _row),
            _BLK((BN, 16), _row), _BLK((1, D), _fix), _BLK((D, OUT), _fix),
            _BLK((1, OUT), _fix),
            pl.BlockSpec((1, 1, BN), lambda j: (j, 0, 0)),
        ],
        out_specs=_BLK((G, OUT), _fix),
        out_shape=jax.ShapeDtypeStruct((G, OUT), jnp.float32),
        scratch_shapes=[
            pltpu.VMEM((G, OUT), jnp.float32),
            pltpu.VMEM((G, 1), jnp.float32),
        ],
    )(a0, a1, g, dinv, b3, w_cls, b_cls, batch3)


def kernel(x, edge_index, batch, W_in, b_in, W1, b1, W2, b2, W3, b3,
           W_cls, b_cls):
    src2 = edge_index[0].reshape(NW, EW)
    dst3 = edge_index[1].reshape(NW, CH, K)
    batch3 = batch.reshape(NB, 1, BN)

    onesD = jnp.ones((K, D), jnp.float32)
    zerosD = jnp.zeros((RPS, D), jnp.float32)

    deg4 = _sc_degree(dst3, onesD, zerosD)
    dega = deg4[0].reshape(N, D)
    degb = deg4[1].reshape(N, D)

    g1, dinv = _tc_encoder(x, W_in, b_in.reshape(1, D), W1, dega, degb)

    acc = _sc_aggregate(g1, src2, dst3, zerosD)
    g2 = _tc_post(acc[0].reshape(N, D), acc[1].reshape(N, D), g1, dinv,
                  b1.reshape(1, D), W2)

    acc = _sc_aggregate(g2, src2, dst3, zerosD)
    g3 = _tc_post(acc[0].reshape(N, D), acc[1].reshape(N, D), g2, dinv,
                  b2.reshape(1, D), W3)

    acc = _sc_aggregate(g3, src2, dst3, zerosD)
    pooled = _tc_final(acc[0].reshape(N, D), acc[1].reshape(N, D), g3, dinv,
                       b3.reshape(1, D), W_cls, b_cls.reshape(1, OUT), batch3)
    return pooled
